# Initial kernel scaffold; baseline (speedup 1.0000x reference)
import jax
import jax.numpy as jnp
from jax.experimental import pallas as pl


def kernel(idx, logits):
    raise NotImplementedError("write your pallas kernel here")


# speedup vs baseline: 1.5789x; 1.5789x over previous
import functools

import jax
import jax.numpy as jnp
from jax import lax
from jax.experimental import pallas as pl
from jax.experimental.pallas import tpu as pltpu
from jax.experimental.pallas import tpu_sc as plsc

_B = 4096
_D = 8192
_NC = 2
_NS = 16
_NW = _NC * _NS
_BW = _B // _NW
_R = 4
_STEPS = _BW // _R

_mesh = plsc.VectorSubcoreMesh(core_axis_name="c", subcore_axis_name="s")


@functools.partial(
    pl.kernel,
    mesh=_mesh,
    out_type=jax.ShapeDtypeStruct((_B, _D), jnp.float32),
    scratch_types=[
        pltpu.VMEM((_STEPS, _R), jnp.int32),
        pltpu.VMEM((_R, _D), jnp.float32),
        pltpu.SemaphoreType.DMA,
    ],
)
def _gather_rows(table_hbm, idx_hbm, out_hbm, idx_v, rows_v, sem):
    wid = lax.axis_index("s") * _NC + lax.axis_index("c")
    base = wid * _BW
    pltpu.sync_copy(idx_hbm.at[wid], idx_v)

    def step(g, carry):
        pltpu.async_copy(table_hbm.at[idx_v.at[g]], rows_v, sem).wait()
        pltpu.sync_copy(rows_v, out_hbm.at[pl.ds(base + g * _R, _R)])
        return carry

    lax.fori_loop(0, _STEPS, step, 0)


def kernel(idx, logits):
    idx3 = idx.astype(jnp.int32).reshape(_NW, _STEPS, _R)
    return _gather_rows(logits, idx3)


# speedup vs baseline: 1.8047x; 1.1430x over previous
import functools

import jax
import jax.numpy as jnp
from jax import lax
from jax.experimental import pallas as pl
from jax.experimental.pallas import tpu as pltpu
from jax.experimental.pallas import tpu_sc as plsc

_B = 4096
_D = 8192
_NC = 2
_NS = 16
_NW = _NC * _NS
_BW = _B // _NW
_R = 4
_STEPS = _BW // _R

_mesh = plsc.VectorSubcoreMesh(core_axis_name="c", subcore_axis_name="s")


@functools.partial(
    pl.kernel,
    mesh=_mesh,
    out_type=jax.ShapeDtypeStruct((_B, _D), jnp.float32),
    scratch_types=[
        pltpu.VMEM((_STEPS, _R), jnp.int32),
        pltpu.VMEM((_R, _D), jnp.float32),
        pltpu.VMEM((_R, _D), jnp.float32),
        pltpu.SemaphoreType.DMA,
        pltpu.SemaphoreType.DMA,
        pltpu.SemaphoreType.DMA,
        pltpu.SemaphoreType.DMA,
    ],
)
def _gather_rows(table_hbm, idx_hbm, out_hbm, idx_v, buf_a, buf_b,
                 gsem_a, gsem_b, ssem_a, ssem_b):
    wid = lax.axis_index("s") * _NC + lax.axis_index("c")
    base = wid * _BW
    pltpu.sync_copy(idx_hbm.at[wid], idx_v)

    bufs = (buf_a, buf_b)
    gsems = (gsem_a, gsem_b)
    ssems = (ssem_a, ssem_b)

    def gather_start(g, p):
        pltpu.async_copy(table_hbm.at[idx_v.at[g]], bufs[p], gsems[p])

    def gather_wait(p):
        pltpu.make_async_copy(table_hbm.at[idx_v.at[0]], bufs[p],
                              gsems[p]).wait()

    def scatter_start(g, p):
        pltpu.async_copy(bufs[p], out_hbm.at[pl.ds(base + g * _R, _R)],
                         ssems[p])

    def scatter_wait(g, p):
        pltpu.make_async_copy(bufs[p], out_hbm.at[pl.ds(base + g * _R, _R)],
                              ssems[p]).wait()

    gather_start(0, 0)
    for g in range(_STEPS):
        p = g % 2
        q = (g + 1) % 2
        gather_wait(p)
        scatter_start(g, p)
        if g + 1 < _STEPS:
            if g >= 1:
                scatter_wait(g - 1, q)
            gather_start(g + 1, q)
    scatter_wait(_STEPS - 2, _STEPS % 2)
    scatter_wait(_STEPS - 1, (_STEPS - 1) % 2)


def kernel(idx, logits):
    idx3 = idx.astype(jnp.int32).reshape(_NW, _STEPS, _R)
    return _gather_rows(logits, idx3)


# speedup vs baseline: 1.8348x; 1.0166x over previous
import functools

import jax
import jax.numpy as jnp
from jax import lax
from jax.experimental import pallas as pl
from jax.experimental.pallas import tpu as pltpu
from jax.experimental.pallas import tpu_sc as plsc

_B = 4096
_D = 8192
_NC = 2
_NS = 16
_NW = _NC * _NS
_BW = _B // _NW
_R = 4
_STEPS = _BW // _R

_mesh = plsc.VectorSubcoreMesh(core_axis_name="c", subcore_axis_name="s")


@functools.partial(
    pl.kernel,
    mesh=_mesh,
    out_type=jax.ShapeDtypeStruct((_B, _D), jnp.float32),
    scratch_types=[
        pltpu.VMEM((_STEPS, _R), jnp.int32),
        pltpu.VMEM((_R, _D), jnp.float32),
        pltpu.VMEM((_R, _D), jnp.float32),
        pltpu.VMEM((_R, _D), jnp.float32),
        pltpu.SemaphoreType.DMA,
        pltpu.SemaphoreType.DMA,
        pltpu.SemaphoreType.DMA,
        pltpu.SemaphoreType.DMA,
        pltpu.SemaphoreType.DMA,
        pltpu.SemaphoreType.DMA,
    ],
)
def _gather_rows(table_hbm, idx_hbm, out_hbm, idx_v, buf_a, buf_b, buf_c,
                 gsem_a, gsem_b, gsem_c, ssem_a, ssem_b, ssem_c):
    wid = lax.axis_index("s") * _NC + lax.axis_index("c")
    base = wid * _BW
    pltpu.sync_copy(idx_hbm.at[wid], idx_v)

    bufs = (buf_a, buf_b, buf_c)
    gsems = (gsem_a, gsem_b, gsem_c)
    ssems = (ssem_a, ssem_b, ssem_c)

    def gather_start(g, p):
        pltpu.async_copy(table_hbm.at[idx_v.at[g]], bufs[p], gsems[p])

    def gather_wait(p):
        pltpu.make_async_copy(table_hbm.at[idx_v.at[0]], bufs[p],
                              gsems[p]).wait()

    def scatter_start(g, p):
        pltpu.async_copy(bufs[p], out_hbm.at[pl.ds(base + g * _R, _R)],
                         ssems[p])

    def scatter_wait(g, p):
        pltpu.make_async_copy(bufs[p], out_hbm.at[pl.ds(base + g * _R, _R)],
                              ssems[p]).wait()

    gather_start(0, 0)
    gather_start(1, 1)
    for g in range(_STEPS):
        p = g % 3
        gather_wait(p)
        scatter_start(g, p)
        if g + 2 < _STEPS:
            if g >= 1:
                scatter_wait(g - 1, (g - 1) % 3)
            gather_start(g + 2, (g + 2) % 3)
    for g in range(_STEPS - 3, _STEPS):
        scatter_wait(g, g % 3)


def kernel(idx, logits):
    idx3 = idx.astype(jnp.int32).reshape(_NW, _STEPS, _R)
    return _gather_rows(logits, idx3)


# speedup vs baseline: 1.8458x; 1.0060x over previous
import functools

import jax
import jax.numpy as jnp
from jax import lax
from jax.experimental import pallas as pl
from jax.experimental.pallas import tpu as pltpu
from jax.experimental.pallas import tpu_sc as plsc

_B = 4096
_D = 8192
_NC = 2
_NS = 16
_NW = _NC * _NS
_BW = _B // _NW
_R = 2
_STEPS = _BW // _R
_NBUF = 7
_LOOK = 4

_mesh = plsc.VectorSubcoreMesh(core_axis_name="c", subcore_axis_name="s")


@functools.partial(
    pl.kernel,
    mesh=_mesh,
    out_type=jax.ShapeDtypeStruct((_B, _D), jnp.float32),
    scratch_types=(
        [pltpu.VMEM((_STEPS, _R), jnp.int32)]
        + [pltpu.VMEM((_R, _D), jnp.float32)] * _NBUF
        + [pltpu.SemaphoreType.DMA] * (2 * _NBUF)
    ),
)
def _gather_rows(table_hbm, idx_hbm, out_hbm, idx_v, *bufs_and_sems):
    bufs = bufs_and_sems[:_NBUF]
    gsems = bufs_and_sems[_NBUF:2 * _NBUF]
    ssems = bufs_and_sems[2 * _NBUF:]
    wid = lax.axis_index("s") * _NC + lax.axis_index("c")
    base = wid * _BW
    pltpu.sync_copy(idx_hbm.at[wid], idx_v)

    def gather_start(g, p):
        pltpu.async_copy(table_hbm.at[idx_v.at[g]], bufs[p], gsems[p])

    def gather_wait(p):
        pltpu.make_async_copy(table_hbm.at[idx_v.at[0]], bufs[p],
                              gsems[p]).wait()

    def scatter_start(g, p):
        pltpu.async_copy(bufs[p], out_hbm.at[pl.ds(base + g * _R, _R)],
                         ssems[p])

    def scatter_wait(g, p):
        pltpu.make_async_copy(bufs[p], out_hbm.at[pl.ds(base + g * _R, _R)],
                              ssems[p]).wait()

    for g in range(_LOOK):
        gather_start(g, g % _NBUF)
    for g in range(_STEPS):
        p = g % _NBUF
        gather_wait(p)
        scatter_start(g, p)
        if g + _LOOK < _STEPS:
            d = g + _LOOK - _NBUF
            if d >= 0:
                scatter_wait(d, d % _NBUF)
            gather_start(g + _LOOK, (g + _LOOK) % _NBUF)
    for g in range(_STEPS - _NBUF, _STEPS):
        scatter_wait(g, g % _NBUF)


def kernel(idx, logits):
    idx3 = idx.astype(jnp.int32).reshape(_NW, _STEPS, _R)
    return _gather_rows(logits, idx3)
